# baseline (speedup 1.0000x reference)
import jax
import jax.numpy as jnp
from jax.experimental import pallas as pl
from jax.experimental.pallas import tpu as pltpu

LANE = 128
SUB = 8
_VMEM_LIMIT = 48 * 1024 * 1024


def _round_up(x, m):
    return ((x + m - 1) // m) * m


def _pad2(a, r, c):
    a = a.astype(jnp.float32)
    return jnp.pad(a, ((0, r - a.shape[0]), (0, c - a.shape[1])))


def _linear_kernel(x_ref, w_ref, b_ref, o_ref):
    o_ref[...] = (jnp.dot(x_ref[...], w_ref[...], preferred_element_type=jnp.float32)
                  + b_ref[...])


def _pallas_linear(x, w, b):
    M, K = x.shape
    N = w.shape[1]
    Mp, Kp, Np = _round_up(M, SUB), _round_up(K, LANE), _round_up(N, LANE)
    out = pl.pallas_call(
        _linear_kernel,
        out_shape=jax.ShapeDtypeStruct((Mp, Np), jnp.float32),
        grid=(1,),
        in_specs=[pl.BlockSpec((Mp, Kp), lambda i: (0, 0)),
                  pl.BlockSpec((Kp, Np), lambda i: (0, 0)),
                  pl.BlockSpec((1, Np), lambda i: (0, 0))],
        out_specs=pl.BlockSpec((Mp, Np), lambda i: (0, 0)),
        compiler_params=pltpu.CompilerParams(
            dimension_semantics=("arbitrary",),
            vmem_limit_bytes=_VMEM_LIMIT),
    )(_pad2(x, Mp, Kp), _pad2(w, Kp, Np), _pad2(b[None, :], 1, Np))
    return out[:M, :N]


def _segscan_tile(val, f, carry_ref, t, is_max):
    fill = -1e30 if is_max else 0.0

    def _op(a, b):
        return jnp.maximum(a, b) if is_max else a + b

    @pl.when(t == 0)
    def _():
        carry_ref[...] = jnp.full(carry_ref.shape, fill, jnp.float32)

    tb, cols = val.shape
    d = 1
    while d < tb:
        val_s = jnp.concatenate(
            [jnp.full((d, cols), fill, jnp.float32), val[:-d, :]], axis=0)
        f_s = jnp.concatenate(
            [jnp.zeros((d, 1), jnp.float32), f[:-d, :]], axis=0)
        val = jnp.where(f > 0.0, val, _op(val, val_s))
        f = jnp.maximum(f, f_s)
        d *= 2
    val = jnp.where(f > 0.0, val, _op(val, carry_ref[...]))
    carry_ref[...] = val[tb - 1:tb, :]
    return val


def _make_edge_mlp_kernel(sub):
    def kernel_fn(cat_ref, flag_ref, w1_ref, b1_ref, w2_ref, b2_ref, o_ref,
                  carry_ref, msg_ref):
        te = cat_ref.shape[0]
        for j in range(te // sub):
            c = cat_ref[j * sub:(j + 1) * sub, :]
            h = jnp.dot(c, w1_ref[...], preferred_element_type=jnp.float32) + b1_ref[...]
            h = jnp.maximum(h, 0.0)
            msg_ref[j * sub:(j + 1) * sub, :] = (
                jnp.dot(h, w2_ref[...], preferred_element_type=jnp.float32) + b2_ref[...])
        o_ref[...] = _segscan_tile(msg_ref[...], flag_ref[...], carry_ref,
                                   pl.program_id(0), is_max=True)

    return kernel_fn


def _pallas_edge_mlp_scanmax(cat, flags, w1, b1, w2, b2, edge_tile=2048, sub=128):
    Ep, Hcat = cat.shape
    Cmid, Cout = w1.shape[1], w2.shape[1]
    Hcatp, Cmidp, Coutp = (_round_up(Hcat, LANE), _round_up(Cmid, LANE),
                           _round_up(Cout, LANE))
    te = min(edge_tile, Ep)
    assert Ep % te == 0 and te % sub == 0
    out = pl.pallas_call(
        _make_edge_mlp_kernel(sub),
        out_shape=jax.ShapeDtypeStruct((Ep, Coutp), jnp.float32),
        grid=(Ep // te,),
        in_specs=[pl.BlockSpec((te, Hcatp), lambda t: (t, 0)),
                  pl.BlockSpec((te, 1), lambda t: (t, 0)),
                  pl.BlockSpec((Hcatp, Cmidp), lambda t: (0, 0)),
                  pl.BlockSpec((1, Cmidp), lambda t: (0, 0)),
                  pl.BlockSpec((Cmidp, Coutp), lambda t: (0, 0)),
                  pl.BlockSpec((1, Coutp), lambda t: (0, 0))],
        out_specs=pl.BlockSpec((te, Coutp), lambda t: (t, 0)),
        scratch_shapes=[pltpu.VMEM((1, Coutp), jnp.float32),
                        pltpu.VMEM((te, Coutp), jnp.float32)],
        compiler_params=pltpu.CompilerParams(
            dimension_semantics=("arbitrary",),
            vmem_limit_bytes=_VMEM_LIMIT),
    )(_pad2(cat, Ep, Hcatp), flags,
      _pad2(w1, Hcatp, Cmidp), _pad2(b1[None, :], 1, Cmidp),
      _pad2(w2, Cmidp, Coutp), _pad2(b2[None, :], 1, Coutp))
    return out


def _make_conv_chain_kernel(n_layers, n_powers, n_kernels, co, neg_slope):
    def kernel_fn(pd_ref, flag_ref, we_ref, be_ref, w_ref, a_ref, m_ref, o_ref,
                  carry_ref):
        ea = (jnp.dot(pd_ref[...], we_ref[...], preferred_element_type=jnp.float32)
              + be_ref[...])
        m = m_ref[...]
        mks = [m[:, k:k + 1] for k in range(n_kernels)]
        for l in range(n_layers):
            h_all = jnp.dot(ea, w_ref[l], preferred_element_type=jnp.float32)
            hps = [h_all[:, 0:co]]
            for i in range(1, n_powers):
                h = h_all[:, i * co:(i + 1) * co]
                h = jnp.where(h > 0, h, neg_slope * h)
                hp = h
                for _ in range(i - 1):
                    hp = hp * h
                hps.append(hp)
            hp_all = jnp.concatenate(hps, axis=1)
            big = jnp.dot(hp_all, a_ref[l], preferred_element_type=jnp.float32)
            norm = mks[0] * big[:, 0:co]
            for k in range(1, n_kernels):
                norm = norm + mks[k] * big[:, k * co:(k + 1) * co]
            ea = norm
        o_ref[...] = _segscan_tile(ea, flag_ref[...], carry_ref,
                                   pl.program_id(0), is_max=False)

    return kernel_fn


def _pallas_conv_chain(pdiff, flags, we, be, w_stack, a_stack, mask,
                       neg_slope=0.1, edge_tile=2048):
    Ep, Cinp = pdiff.shape
    L, co, PCo = w_stack.shape
    KCo = a_stack.shape[2]
    P = PCo // co
    K = KCo // co
    te = min(edge_tile, Ep)
    assert Ep % te == 0

    flops = 2 * Ep * L * (co * PCo + PCo * KCo)
    bytes_accessed = 4 * (Ep * Cinp + L * (co * PCo + PCo * KCo)
                          + Ep * LANE + Ep * co)

    return pl.pallas_call(
        _make_conv_chain_kernel(L, P, K, co, neg_slope),
        out_shape=jax.ShapeDtypeStruct((Ep, co), jnp.float32),
        grid=(Ep // te,),
        in_specs=[pl.BlockSpec((te, Cinp), lambda t: (t, 0)),
                  pl.BlockSpec((te, 1), lambda t: (t, 0)),
                  pl.BlockSpec((Cinp, co), lambda t: (0, 0)),
                  pl.BlockSpec((1, co), lambda t: (0, 0)),
                  pl.BlockSpec((L, co, PCo), lambda t: (0, 0, 0)),
                  pl.BlockSpec((L, PCo, KCo), lambda t: (0, 0, 0)),
                  pl.BlockSpec((te, LANE), lambda t: (t, 0))],
        out_specs=pl.BlockSpec((te, co), lambda t: (t, 0)),
        scratch_shapes=[pltpu.VMEM((1, co), jnp.float32)],
        compiler_params=pltpu.CompilerParams(
            dimension_semantics=("arbitrary",),
            vmem_limit_bytes=_VMEM_LIMIT),
        cost_estimate=pl.CostEstimate(flops=flops, transcendentals=0,
                                      bytes_accessed=bytes_accessed),
    )(pdiff, flags, we, be, w_stack, a_stack, mask)


def _kmeans(x_feat, n_clusters, key, iters=25):
    n_points = x_feat.shape[0]
    init_idx = jax.random.randint(key, (n_clusters,), 0, n_points)
    centroids = x_feat[init_idx]
    x_sq = jnp.sum(x_feat * x_feat, axis=1, keepdims=True)

    def dists(cent):
        return x_sq - 2.0 * x_feat @ cent.T + jnp.sum(cent * cent, axis=1)[None, :]

    def body(_, cent):
        assign = jnp.argmin(dists(cent), axis=1)
        onehot = jax.nn.one_hot(assign, n_clusters, dtype=jnp.float32)
        counts = onehot.sum(axis=0)
        sums = onehot.T @ x_feat
        new_cent = sums / jnp.maximum(counts, 1.0)[:, None]
        return jnp.where(counts[:, None] > 0, new_cent, cent)

    centroids = jax.lax.fori_loop(0, iters, body, centroids)
    return jnp.argmin(dists(centroids), axis=1)


def kernel(x, pos, edge_index, kmeans_key_data,
           lin_similar_w, lin_similar_b, lin_x_w, lin_x_b, lin_edge_w, lin_edge_b,
           ec_w1, ec_b1, ec_w2, ec_b2,
           conv1_W, conv1_alpha, conv2_W, conv2_alpha,
           conv4_W, conv4_alpha, conv5_W, conv5_alpha):
    kmeans_key = jax.random.wrap_key_data(kmeans_key_data)
    num_kernels = conv1_alpha.shape[0]
    N = x.shape[0]
    src, tgt = edge_index[0], edge_index[1]
    E = src.shape[0]
    Ep = _round_up(E, LANE)
    Coutp = _round_up(conv5_W.shape[2], LANE)

    perm = jnp.argsort(tgt)
    src_s = src[perm]
    tgt_s = tgt[perm]
    tgt_sp = jnp.pad(tgt_s, (0, Ep - E), constant_values=N)
    prev = jnp.concatenate([jnp.full((1,), -1, tgt_sp.dtype), tgt_sp[:-1]])
    flags = (tgt_sp != prev).astype(jnp.float32)[:, None]
    counts = jnp.zeros((N,), jnp.int32).at[tgt].add(1)
    seg_end = jnp.cumsum(counts) - 1
    has_in = counts > 0
    deg = jnp.zeros((N,), jnp.float32).at[src].add(1.0)

    hid = lin_x_w.shape[1]
    xin = jnp.concatenate([x, pos], axis=1).astype(jnp.float32)
    wx_aug = jnp.concatenate([lin_x_w, jnp.zeros((2, hid), jnp.float32)], axis=0)
    w_fused = jnp.concatenate([wx_aug, lin_similar_w], axis=1)
    b_fused = jnp.concatenate([lin_x_b, lin_similar_b])
    fused = _pallas_linear(xin, w_fused, b_fused)
    x_similar = jax.nn.relu(fused[:, hid:])

    hidw = x_similar.shape[1]
    node_tab = jnp.concatenate(
        [x_similar, pos.astype(jnp.float32), deg[:, None]], axis=1)
    g_tgt = jnp.take(node_tab, tgt_s, axis=0)
    g_src = jnp.take(node_tab, src_s, axis=0)

    cat = jnp.concatenate([g_tgt[:, :hidw], g_src[:, :hidw] - g_tgt[:, :hidw]],
                          axis=1)
    scan_max = _pallas_edge_mlp_scanmax(cat, flags, ec_w1, ec_b1, ec_w2, ec_b2)
    gathered = jnp.take(scan_max, seg_end, axis=0)[:, :ec_w2.shape[1]]
    x_similar = jnp.where(has_in[:, None], gathered, 0.0)
    x_similar = jax.nn.relu(x_similar)
    cluster = _kmeans(x_similar, num_kernels, kmeans_key)

    krange = jnp.arange(num_kernels, dtype=cluster.dtype)[None, :]
    c_src = jnp.take(cluster, src_s)
    c_tgt = jnp.take(cluster, tgt_s)
    edge_mask = ((c_src[:, None] == krange)
                 & (c_tgt[:, None] == krange)).astype(jnp.float32)
    inv_deg_s = 1.0 / g_src[:, hidw + 2]
    mask_scaled = edge_mask * inv_deg_s[:, None]
    mask_scaled = jnp.pad(mask_scaled, ((0, Ep - E), (0, LANE - num_kernels)))

    pdiff = _pad2(g_tgt[:, hidw:hidw + 2] - g_src[:, hidw:hidw + 2], Ep, LANE)
    we = _pad2(lin_edge_w, LANE, Coutp)
    be = _pad2(lin_edge_b[None, :], 1, Coutp)
    assert conv1_W.shape[1] == LANE and Coutp == conv5_W.shape[2]
    w_stack = jnp.stack(
        [W.transpose(1, 0, 2).reshape(W.shape[1], -1)
         for W in (conv1_W, conv2_W, conv4_W, conv5_W)])
    a_stack = jnp.stack(
        [al.transpose(1, 3, 0, 2).reshape(al.shape[1] * al.shape[3], -1)
         for al in (conv1_alpha, conv2_alpha, conv4_alpha, conv5_alpha)])
    scan_sum = _pallas_conv_chain(pdiff, flags, we, be, w_stack, a_stack,
                                  mask_scaled)

    e_nodes = jnp.take(scan_sum, seg_end, axis=0)[:, :conv5_W.shape[2]]
    return jnp.where(has_in[:, None], e_nodes, 0.0)


# speedup vs baseline: 1.0078x; 1.0078x over previous
import jax
import jax.numpy as jnp
from jax.experimental import pallas as pl
from jax.experimental.pallas import tpu as pltpu

LANE = 128
SUB = 8
_VMEM_LIMIT = 48 * 1024 * 1024


def _round_up(x, m):
    return ((x + m - 1) // m) * m


def _pad2(a, r, c):
    a = a.astype(jnp.float32)
    return jnp.pad(a, ((0, r - a.shape[0]), (0, c - a.shape[1])))


def _linear_kernel(x_ref, w_ref, b_ref, o_ref):
    o_ref[...] = (jnp.dot(x_ref[...], w_ref[...], preferred_element_type=jnp.float32)
                  + b_ref[...])


def _pallas_linear(x, w, b):
    M, K = x.shape
    N = w.shape[1]
    Mp, Kp, Np = _round_up(M, SUB), _round_up(K, LANE), _round_up(N, LANE)
    out = pl.pallas_call(
        _linear_kernel,
        out_shape=jax.ShapeDtypeStruct((Mp, Np), jnp.float32),
        grid=(1,),
        in_specs=[pl.BlockSpec((Mp, Kp), lambda i: (0, 0)),
                  pl.BlockSpec((Kp, Np), lambda i: (0, 0)),
                  pl.BlockSpec((1, Np), lambda i: (0, 0))],
        out_specs=pl.BlockSpec((Mp, Np), lambda i: (0, 0)),
        compiler_params=pltpu.CompilerParams(
            dimension_semantics=("arbitrary",),
            vmem_limit_bytes=_VMEM_LIMIT),
    )(_pad2(x, Mp, Kp), _pad2(w, Kp, Np), _pad2(b[None, :], 1, Np))
    return out[:M, :N]


def _segscan_tile(val, f, carry_ref, t, is_max):
    fill = -1e30 if is_max else 0.0

    def _op(a, b):
        return jnp.maximum(a, b) if is_max else a + b

    @pl.when(t == 0)
    def _():
        carry_ref[...] = jnp.full(carry_ref.shape, fill, jnp.float32)

    tb, cols = val.shape
    d = 1
    while d < tb:
        val_s = jnp.concatenate(
            [jnp.full((d, cols), fill, jnp.float32), val[:-d, :]], axis=0)
        f_s = jnp.concatenate(
            [jnp.zeros((d, 1), jnp.float32), f[:-d, :]], axis=0)
        val = jnp.where(f > 0.0, val, _op(val, val_s))
        f = jnp.maximum(f, f_s)
        d *= 2
    val = jnp.where(f > 0.0, val, _op(val, carry_ref[...]))
    carry_ref[...] = val[tb - 1:tb, :]
    return val


def _make_edge_mlp_kernel(sub):
    def kernel_fn(cat_ref, flag_ref, w1_ref, b1_ref, w2_ref, b2_ref, o_ref,
                  carry_ref, msg_ref):
        te = cat_ref.shape[0]
        for j in range(te // sub):
            c = cat_ref[j * sub:(j + 1) * sub, :]
            h = jnp.dot(c, w1_ref[...], preferred_element_type=jnp.float32) + b1_ref[...]
            h = jnp.maximum(h, 0.0)
            msg_ref[j * sub:(j + 1) * sub, :] = (
                jnp.dot(h, w2_ref[...], preferred_element_type=jnp.float32) + b2_ref[...])
        o_ref[...] = _segscan_tile(msg_ref[...], flag_ref[...], carry_ref,
                                   pl.program_id(0), is_max=True)

    return kernel_fn


def _pallas_edge_mlp_scanmax(cat, flags, w1, b1, w2, b2, edge_tile=2048, sub=128):
    Ep, Hcat = cat.shape
    Cmid, Cout = w1.shape[1], w2.shape[1]
    Hcatp, Cmidp, Coutp = (_round_up(Hcat, LANE), _round_up(Cmid, LANE),
                           _round_up(Cout, LANE))
    te = min(edge_tile, Ep)
    assert Ep % te == 0 and te % sub == 0
    out = pl.pallas_call(
        _make_edge_mlp_kernel(sub),
        out_shape=jax.ShapeDtypeStruct((Ep, Coutp), jnp.float32),
        grid=(Ep // te,),
        in_specs=[pl.BlockSpec((te, Hcatp), lambda t: (t, 0)),
                  pl.BlockSpec((te, 1), lambda t: (t, 0)),
                  pl.BlockSpec((Hcatp, Cmidp), lambda t: (0, 0)),
                  pl.BlockSpec((1, Cmidp), lambda t: (0, 0)),
                  pl.BlockSpec((Cmidp, Coutp), lambda t: (0, 0)),
                  pl.BlockSpec((1, Coutp), lambda t: (0, 0))],
        out_specs=pl.BlockSpec((te, Coutp), lambda t: (t, 0)),
        scratch_shapes=[pltpu.VMEM((1, Coutp), jnp.float32),
                        pltpu.VMEM((te, Coutp), jnp.float32)],
        compiler_params=pltpu.CompilerParams(
            dimension_semantics=("arbitrary",),
            vmem_limit_bytes=_VMEM_LIMIT),
    )(_pad2(cat, Ep, Hcatp), flags,
      _pad2(w1, Hcatp, Cmidp), _pad2(b1[None, :], 1, Cmidp),
      _pad2(w2, Cmidp, Coutp), _pad2(b2[None, :], 1, Coutp))
    return out


def _make_conv_chain_kernel(n_layers, n_powers, n_kernels, co, neg_slope):
    def kernel_fn(pd_ref, flag_ref, we_ref, be_ref, w_ref, a_ref, m_ref, o_ref,
                  carry_ref):
        ea = (jnp.dot(pd_ref[...], we_ref[...], preferred_element_type=jnp.float32)
              + be_ref[...])
        m = m_ref[...]
        mks = [m[:, k:k + 1] for k in range(n_kernels)]
        for l in range(n_layers):
            h_all = jnp.dot(ea.astype(jnp.bfloat16), w_ref[l],
                            preferred_element_type=jnp.float32)
            hps = [h_all[:, 0:co]]
            for i in range(1, n_powers):
                h = h_all[:, i * co:(i + 1) * co]
                h = jnp.where(h > 0, h, neg_slope * h)
                hp = h
                for _ in range(i - 1):
                    hp = hp * h
                hps.append(hp)
            hp_all = jnp.concatenate(hps, axis=1)
            big = jnp.dot(hp_all.astype(jnp.bfloat16), a_ref[l],
                          preferred_element_type=jnp.float32)
            norm = mks[0] * big[:, 0:co]
            for k in range(1, n_kernels):
                norm = norm + mks[k] * big[:, k * co:(k + 1) * co]
            ea = norm
        o_ref[...] = _segscan_tile(ea, flag_ref[...], carry_ref,
                                   pl.program_id(0), is_max=False)

    return kernel_fn


def _pallas_conv_chain(pdiff, flags, we, be, w_stack, a_stack, mask,
                       neg_slope=0.1, edge_tile=2048):
    Ep, Cinp = pdiff.shape
    L, co, PCo = w_stack.shape
    KCo = a_stack.shape[2]
    P = PCo // co
    K = KCo // co
    te = min(edge_tile, Ep)
    assert Ep % te == 0

    flops = 2 * Ep * L * (co * PCo + PCo * KCo)
    bytes_accessed = 4 * (Ep * Cinp + L * (co * PCo + PCo * KCo)
                          + Ep * LANE + Ep * co)

    return pl.pallas_call(
        _make_conv_chain_kernel(L, P, K, co, neg_slope),
        out_shape=jax.ShapeDtypeStruct((Ep, co), jnp.float32),
        grid=(Ep // te,),
        in_specs=[pl.BlockSpec((te, Cinp), lambda t: (t, 0)),
                  pl.BlockSpec((te, 1), lambda t: (t, 0)),
                  pl.BlockSpec((Cinp, co), lambda t: (0, 0)),
                  pl.BlockSpec((1, co), lambda t: (0, 0)),
                  pl.BlockSpec((L, co, PCo), lambda t: (0, 0, 0)),
                  pl.BlockSpec((L, PCo, KCo), lambda t: (0, 0, 0)),
                  pl.BlockSpec((te, LANE), lambda t: (t, 0))],
        out_specs=pl.BlockSpec((te, co), lambda t: (t, 0)),
        scratch_shapes=[pltpu.VMEM((1, co), jnp.float32)],
        compiler_params=pltpu.CompilerParams(
            dimension_semantics=("arbitrary",),
            vmem_limit_bytes=_VMEM_LIMIT),
        cost_estimate=pl.CostEstimate(flops=flops, transcendentals=0,
                                      bytes_accessed=bytes_accessed),
    )(pdiff, flags, we, be, w_stack, a_stack, mask)


def _kmeans(x_feat, n_clusters, key, iters=25):
    n_points = x_feat.shape[0]
    init_idx = jax.random.randint(key, (n_clusters,), 0, n_points)
    centroids = x_feat[init_idx]
    x_sq = jnp.sum(x_feat * x_feat, axis=1, keepdims=True)

    def dists(cent):
        return x_sq - 2.0 * x_feat @ cent.T + jnp.sum(cent * cent, axis=1)[None, :]

    def body(_, cent):
        assign = jnp.argmin(dists(cent), axis=1)
        onehot = jax.nn.one_hot(assign, n_clusters, dtype=jnp.float32)
        counts = onehot.sum(axis=0)
        sums = onehot.T @ x_feat
        new_cent = sums / jnp.maximum(counts, 1.0)[:, None]
        return jnp.where(counts[:, None] > 0, new_cent, cent)

    centroids = jax.lax.fori_loop(0, iters, body, centroids)
    return jnp.argmin(dists(centroids), axis=1)


def kernel(x, pos, edge_index, kmeans_key_data,
           lin_similar_w, lin_similar_b, lin_x_w, lin_x_b, lin_edge_w, lin_edge_b,
           ec_w1, ec_b1, ec_w2, ec_b2,
           conv1_W, conv1_alpha, conv2_W, conv2_alpha,
           conv4_W, conv4_alpha, conv5_W, conv5_alpha):
    kmeans_key = jax.random.wrap_key_data(kmeans_key_data)
    num_kernels = conv1_alpha.shape[0]
    N = x.shape[0]
    src, tgt = edge_index[0], edge_index[1]
    E = src.shape[0]
    Ep = _round_up(E, LANE)
    Coutp = _round_up(conv5_W.shape[2], LANE)

    perm = jnp.argsort(tgt)
    src_s = src[perm]
    tgt_s = tgt[perm]
    tgt_sp = jnp.pad(tgt_s, (0, Ep - E), constant_values=N)
    prev = jnp.concatenate([jnp.full((1,), -1, tgt_sp.dtype), tgt_sp[:-1]])
    flags = (tgt_sp != prev).astype(jnp.float32)[:, None]
    counts = jnp.zeros((N,), jnp.int32).at[tgt].add(1)
    seg_end = jnp.cumsum(counts) - 1
    has_in = counts > 0
    deg = jnp.zeros((N,), jnp.float32).at[src].add(1.0)

    hid = lin_x_w.shape[1]
    xin = jnp.concatenate([x, pos], axis=1).astype(jnp.float32)
    wx_aug = jnp.concatenate([lin_x_w, jnp.zeros((2, hid), jnp.float32)], axis=0)
    w_fused = jnp.concatenate([wx_aug, lin_similar_w], axis=1)
    b_fused = jnp.concatenate([lin_x_b, lin_similar_b])
    fused = _pallas_linear(xin, w_fused, b_fused)
    x_similar = jax.nn.relu(fused[:, hid:])

    hidw = x_similar.shape[1]
    node_tab = jnp.concatenate(
        [x_similar, pos.astype(jnp.float32), deg[:, None]], axis=1)
    g_tgt = jnp.take(node_tab, tgt_s, axis=0)
    g_src = jnp.take(node_tab, src_s, axis=0)

    cat = jnp.concatenate([g_tgt[:, :hidw], g_src[:, :hidw] - g_tgt[:, :hidw]],
                          axis=1)
    scan_max = _pallas_edge_mlp_scanmax(cat, flags, ec_w1, ec_b1, ec_w2, ec_b2)
    gathered = jnp.take(scan_max, seg_end, axis=0)[:, :ec_w2.shape[1]]
    x_similar = jnp.where(has_in[:, None], gathered, 0.0)
    x_similar = jax.nn.relu(x_similar)
    cluster = _kmeans(x_similar, num_kernels, kmeans_key)

    krange = jnp.arange(num_kernels, dtype=cluster.dtype)[None, :]
    c_src = jnp.take(cluster, src_s)
    c_tgt = jnp.take(cluster, tgt_s)
    edge_mask = ((c_src[:, None] == krange)
                 & (c_tgt[:, None] == krange)).astype(jnp.float32)
    inv_deg_s = 1.0 / g_src[:, hidw + 2]
    mask_scaled = edge_mask * inv_deg_s[:, None]
    mask_scaled = jnp.pad(mask_scaled, ((0, Ep - E), (0, LANE - num_kernels)))

    pdiff = _pad2(g_tgt[:, hidw:hidw + 2] - g_src[:, hidw:hidw + 2], Ep, LANE)
    we = _pad2(lin_edge_w, LANE, Coutp)
    be = _pad2(lin_edge_b[None, :], 1, Coutp)
    assert conv1_W.shape[1] == LANE and Coutp == conv5_W.shape[2]
    w_stack = jnp.stack(
        [W.transpose(1, 0, 2).reshape(W.shape[1], -1)
         for W in (conv1_W, conv2_W, conv4_W, conv5_W)]).astype(jnp.bfloat16)
    a_stack = jnp.stack(
        [al.transpose(1, 3, 0, 2).reshape(al.shape[1] * al.shape[3], -1)
         for al in (conv1_alpha, conv2_alpha, conv4_alpha, conv5_alpha)
         ]).astype(jnp.bfloat16)
    scan_sum = _pallas_conv_chain(pdiff, flags, we, be, w_stack, a_stack,
                                  mask_scaled)

    e_nodes = jnp.take(scan_sum, seg_end, axis=0)[:, :conv5_W.shape[2]]
    return jnp.where(has_in[:, None], e_nodes, 0.0)


# speedup vs baseline: 1.0832x; 1.0748x over previous
import jax
import jax.numpy as jnp
from jax.experimental import pallas as pl
from jax.experimental.pallas import tpu as pltpu

LANE = 128
SUB = 8
_VMEM_LIMIT = 48 * 1024 * 1024


def _round_up(x, m):
    return ((x + m - 1) // m) * m


def _pad2(a, r, c):
    a = a.astype(jnp.float32)
    return jnp.pad(a, ((0, r - a.shape[0]), (0, c - a.shape[1])))


def _linear_kernel(x_ref, w_ref, b_ref, o_ref):
    o_ref[...] = (jnp.dot(x_ref[...], w_ref[...], preferred_element_type=jnp.float32)
                  + b_ref[...])


def _pallas_linear(x, w, b):
    M, K = x.shape
    N = w.shape[1]
    Mp, Kp, Np = _round_up(M, SUB), _round_up(K, LANE), _round_up(N, LANE)
    out = pl.pallas_call(
        _linear_kernel,
        out_shape=jax.ShapeDtypeStruct((Mp, Np), jnp.float32),
        grid=(1,),
        in_specs=[pl.BlockSpec((Mp, Kp), lambda i: (0, 0)),
                  pl.BlockSpec((Kp, Np), lambda i: (0, 0)),
                  pl.BlockSpec((1, Np), lambda i: (0, 0))],
        out_specs=pl.BlockSpec((Mp, Np), lambda i: (0, 0)),
        compiler_params=pltpu.CompilerParams(
            dimension_semantics=("arbitrary",),
            vmem_limit_bytes=_VMEM_LIMIT),
    )(_pad2(x, Mp, Kp), _pad2(w, Kp, Np), _pad2(b[None, :], 1, Np))
    return out[:M, :N]


def _segscan_tile(val, f, carry_ref, t, is_max):
    fill = -1e30 if is_max else 0.0

    def _op(a, b):
        return jnp.maximum(a, b) if is_max else a + b

    @pl.when(t == 0)
    def _():
        carry_ref[...] = jnp.full(carry_ref.shape, fill, jnp.float32)

    tb, cols = val.shape
    d = 1
    while d < tb:
        val_s = jnp.concatenate(
            [jnp.full((d, cols), fill, jnp.float32), val[:-d, :]], axis=0)
        f_s = jnp.concatenate(
            [jnp.zeros((d, 1), jnp.float32), f[:-d, :]], axis=0)
        val = jnp.where(f > 0.0, val, _op(val, val_s))
        f = jnp.maximum(f, f_s)
        d *= 2
    val = jnp.where(f > 0.0, val, _op(val, carry_ref[...]))
    carry_ref[...] = val[tb - 1:tb, :]
    return val


def _make_edge_mlp_kernel(sub, hidp):
    def kernel_fn(z_ref, gs_ref, flag_ref, w1_ref, b1_ref, w2_ref, b2_ref,
                  o_ref, pd_ref, carry_z, carry_m, msg_ref):
        t = pl.program_id(0)
        f = flag_ref[...]
        zt = _segscan_tile(z_ref[...], f, carry_z, t, is_max=True)
        gs = gs_ref[...]
        x_t = zt[:, 0:hidp]
        x_s = gs[:, 0:hidp]
        pd = zt[:, hidp:hidp + 2] - gs[:, hidp:hidp + 2]
        pd_ref[...] = jnp.concatenate(
            [pd, jnp.zeros((pd.shape[0], pd_ref.shape[1] - 2), jnp.float32)], axis=1)
        te = zt.shape[0]
        for j in range(te // sub):
            sl = slice(j * sub, (j + 1) * sub)
            c = jnp.concatenate([x_t[sl], x_s[sl] - x_t[sl]], axis=1)
            h = jnp.dot(c, w1_ref[...], preferred_element_type=jnp.float32) + b1_ref[...]
            h = jnp.maximum(h, 0.0)
            msg_ref[sl, :] = (
                jnp.dot(h, w2_ref[...], preferred_element_type=jnp.float32) + b2_ref[...])
        o_ref[...] = _segscan_tile(msg_ref[...], f, carry_m, t, is_max=True)

    return kernel_fn


def _pallas_edge_mlp_scanmax(z_tab, g_src, flags, hidp, w1, b1, w2, b2,
                             edge_tile=2048, sub=128):
    Ep, W = z_tab.shape
    Hcat = 2 * hidp
    Cmid, Cout = w1.shape[1], w2.shape[1]
    Hcatp, Cmidp, Coutp = (_round_up(Hcat, LANE), _round_up(Cmid, LANE),
                           _round_up(Cout, LANE))
    te = min(edge_tile, Ep)
    assert Ep % te == 0 and te % sub == 0
    out, pdiff = pl.pallas_call(
        _make_edge_mlp_kernel(sub, hidp),
        out_shape=(jax.ShapeDtypeStruct((Ep, Coutp), jnp.float32),
                   jax.ShapeDtypeStruct((Ep, LANE), jnp.float32)),
        grid=(Ep // te,),
        in_specs=[pl.BlockSpec((te, W), lambda t: (t, 0)),
                  pl.BlockSpec((te, W), lambda t: (t, 0)),
                  pl.BlockSpec((te, 1), lambda t: (t, 0)),
                  pl.BlockSpec((Hcatp, Cmidp), lambda t: (0, 0)),
                  pl.BlockSpec((1, Cmidp), lambda t: (0, 0)),
                  pl.BlockSpec((Cmidp, Coutp), lambda t: (0, 0)),
                  pl.BlockSpec((1, Coutp), lambda t: (0, 0))],
        out_specs=(pl.BlockSpec((te, Coutp), lambda t: (t, 0)),
                   pl.BlockSpec((te, LANE), lambda t: (t, 0))),
        scratch_shapes=[pltpu.VMEM((1, W), jnp.float32),
                        pltpu.VMEM((1, Coutp), jnp.float32),
                        pltpu.VMEM((te, Coutp), jnp.float32)],
        compiler_params=pltpu.CompilerParams(
            dimension_semantics=("arbitrary",),
            vmem_limit_bytes=_VMEM_LIMIT),
    )(z_tab, g_src, flags,
      _pad2(w1, Hcatp, Cmidp), _pad2(b1[None, :], 1, Cmidp),
      _pad2(w2, Cmidp, Coutp), _pad2(b2[None, :], 1, Coutp))
    return out, pdiff


def _make_conv_chain_kernel(n_layers, n_powers, n_kernels, co, neg_slope):
    def kernel_fn(pd_ref, flag_ref, we_ref, be_ref, w_ref, a_ref, m_ref, o_ref,
                  carry_ref):
        ea = (jnp.dot(pd_ref[...], we_ref[...], preferred_element_type=jnp.float32)
              + be_ref[...])
        m = m_ref[...]
        mks = [m[:, k:k + 1] for k in range(n_kernels)]
        for l in range(n_layers):
            h_all = jnp.dot(ea.astype(jnp.bfloat16), w_ref[l],
                            preferred_element_type=jnp.float32)
            hps = [h_all[:, 0:co]]
            for i in range(1, n_powers):
                h = h_all[:, i * co:(i + 1) * co]
                h = jnp.where(h > 0, h, neg_slope * h)
                hp = h
                for _ in range(i - 1):
                    hp = hp * h
                hps.append(hp)
            hp_all = jnp.concatenate(hps, axis=1)
            big = jnp.dot(hp_all.astype(jnp.bfloat16), a_ref[l],
                          preferred_element_type=jnp.float32)
            norm = mks[0] * big[:, 0:co]
            for k in range(1, n_kernels):
                norm = norm + mks[k] * big[:, k * co:(k + 1) * co]
            ea = norm
        o_ref[...] = _segscan_tile(ea, flag_ref[...], carry_ref,
                                   pl.program_id(0), is_max=False)

    return kernel_fn


def _pallas_conv_chain(pdiff, flags, we, be, w_stack, a_stack, mask,
                       neg_slope=0.1, edge_tile=2048):
    Ep, Cinp = pdiff.shape
    L, co, PCo = w_stack.shape
    KCo = a_stack.shape[2]
    P = PCo // co
    K = KCo // co
    te = min(edge_tile, Ep)
    assert Ep % te == 0

    flops = 2 * Ep * L * (co * PCo + PCo * KCo)
    bytes_accessed = 4 * (Ep * Cinp + L * (co * PCo + PCo * KCo)
                          + Ep * LANE + Ep * co)

    return pl.pallas_call(
        _make_conv_chain_kernel(L, P, K, co, neg_slope),
        out_shape=jax.ShapeDtypeStruct((Ep, co), jnp.float32),
        grid=(Ep // te,),
        in_specs=[pl.BlockSpec((te, Cinp), lambda t: (t, 0)),
                  pl.BlockSpec((te, 1), lambda t: (t, 0)),
                  pl.BlockSpec((Cinp, co), lambda t: (0, 0)),
                  pl.BlockSpec((1, co), lambda t: (0, 0)),
                  pl.BlockSpec((L, co, PCo), lambda t: (0, 0, 0)),
                  pl.BlockSpec((L, PCo, KCo), lambda t: (0, 0, 0)),
                  pl.BlockSpec((te, LANE), lambda t: (t, 0))],
        out_specs=pl.BlockSpec((te, co), lambda t: (t, 0)),
        scratch_shapes=[pltpu.VMEM((1, co), jnp.float32)],
        compiler_params=pltpu.CompilerParams(
            dimension_semantics=("arbitrary",),
            vmem_limit_bytes=_VMEM_LIMIT),
        cost_estimate=pl.CostEstimate(flops=flops, transcendentals=0,
                                      bytes_accessed=bytes_accessed),
    )(pdiff, flags, we, be, w_stack, a_stack, mask)


def _kmeans(x_feat, n_clusters, key, iters=25):
    n_points = x_feat.shape[0]
    init_idx = jax.random.randint(key, (n_clusters,), 0, n_points)
    centroids = x_feat[init_idx]
    x_sq = jnp.sum(x_feat * x_feat, axis=1, keepdims=True)

    def dists(cent):
        return x_sq - 2.0 * x_feat @ cent.T + jnp.sum(cent * cent, axis=1)[None, :]

    def body(_, cent):
        assign = jnp.argmin(dists(cent), axis=1)
        onehot = jax.nn.one_hot(assign, n_clusters, dtype=jnp.float32)
        counts = onehot.sum(axis=0)
        sums = onehot.T @ x_feat
        new_cent = sums / jnp.maximum(counts, 1.0)[:, None]
        return jnp.where(counts[:, None] > 0, new_cent, cent)

    centroids = jax.lax.fori_loop(0, iters, body, centroids)
    return jnp.argmin(dists(centroids), axis=1)


def kernel(x, pos, edge_index, kmeans_key_data,
           lin_similar_w, lin_similar_b, lin_x_w, lin_x_b, lin_edge_w, lin_edge_b,
           ec_w1, ec_b1, ec_w2, ec_b2,
           conv1_W, conv1_alpha, conv2_W, conv2_alpha,
           conv4_W, conv4_alpha, conv5_W, conv5_alpha):
    kmeans_key = jax.random.wrap_key_data(kmeans_key_data)
    num_kernels = conv1_alpha.shape[0]
    N = x.shape[0]
    src, tgt = edge_index[0], edge_index[1]
    E = src.shape[0]
    Ep = _round_up(E, LANE)
    Coutp = _round_up(conv5_W.shape[2], LANE)

    perm = jnp.argsort(tgt)
    src_s = src[perm]
    tgt_s = tgt[perm]
    tgt_sp = jnp.pad(tgt_s, (0, Ep - E), constant_values=N)
    prev = jnp.concatenate([jnp.full((1,), -1, tgt_sp.dtype), tgt_sp[:-1]])
    flags = (tgt_sp != prev).astype(jnp.float32)[:, None]
    counts = jnp.zeros((N,), jnp.int32).at[tgt].add(1)
    seg_end = jnp.cumsum(counts) - 1
    has_in = counts > 0
    deg = jnp.zeros((N,), jnp.float32).at[src].add(1.0)

    hid = lin_x_w.shape[1]
    xin = jnp.concatenate([x, pos], axis=1).astype(jnp.float32)
    wx_aug = jnp.concatenate([lin_x_w, jnp.zeros((2, hid), jnp.float32)], axis=0)
    w_fused = jnp.concatenate([wx_aug, lin_similar_w], axis=1)
    b_fused = jnp.concatenate([lin_x_b, lin_similar_b])
    fused = _pallas_linear(xin, w_fused, b_fused)
    x_similar = jax.nn.relu(fused[:, hid:])

    hidw = x_similar.shape[1]
    tabw = _round_up(hidw + 3, LANE)
    node_tab = jnp.concatenate(
        [x_similar, pos.astype(jnp.float32), deg[:, None],
         jnp.zeros((N, tabw - hidw - 3), jnp.float32)], axis=1)
    g_src = jnp.pad(jnp.take(node_tab, src_s, axis=0), ((0, Ep - E), (0, 0)))
    seg_start = seg_end - counts + 1
    z_idx = jnp.where(has_in, seg_start, Ep)
    z_tab = jnp.full((Ep, tabw), -1e30, jnp.float32).at[z_idx].set(
        node_tab, mode="drop")

    scan_max, pdiff = _pallas_edge_mlp_scanmax(
        z_tab, g_src, flags, hidw, ec_w1, ec_b1, ec_w2, ec_b2)
    gathered = jnp.take(scan_max, seg_end, axis=0)[:, :ec_w2.shape[1]]
    x_similar = jnp.where(has_in[:, None], gathered, 0.0)
    x_similar = jax.nn.relu(x_similar)
    cluster = _kmeans(x_similar, num_kernels, kmeans_key)

    krange = jnp.arange(num_kernels, dtype=cluster.dtype)[None, :]
    c_src = jnp.take(cluster, src_s)
    c_tgt = jnp.take(cluster, tgt_s)
    edge_mask = ((c_src[:, None] == krange)
                 & (c_tgt[:, None] == krange)).astype(jnp.float32)
    inv_deg_s = 1.0 / g_src[:E, hidw + 2]
    mask_scaled = edge_mask * inv_deg_s[:, None]
    mask_scaled = jnp.pad(mask_scaled, ((0, Ep - E), (0, LANE - num_kernels)))

    we = _pad2(lin_edge_w, LANE, Coutp)
    be = _pad2(lin_edge_b[None, :], 1, Coutp)
    assert conv1_W.shape[1] == LANE and Coutp == conv5_W.shape[2]
    w_stack = jnp.stack(
        [W.transpose(1, 0, 2).reshape(W.shape[1], -1)
         for W in (conv1_W, conv2_W, conv4_W, conv5_W)]).astype(jnp.bfloat16)
    a_stack = jnp.stack(
        [al.transpose(1, 3, 0, 2).reshape(al.shape[1] * al.shape[3], -1)
         for al in (conv1_alpha, conv2_alpha, conv4_alpha, conv5_alpha)
         ]).astype(jnp.bfloat16)
    scan_sum = _pallas_conv_chain(pdiff, flags, we, be, w_stack, a_stack,
                                  mask_scaled)

    e_nodes = jnp.take(scan_sum, seg_end, axis=0)[:, :conv5_W.shape[2]]
    return jnp.where(has_in[:, None], e_nodes, 0.0)
